# Initial kernel scaffold; baseline (speedup 1.0000x reference)
import jax
import jax.numpy as jnp
from jax.experimental import pallas as pl


def kernel(x, edge_index, edge_attr, Wp0, bp0, g0, mu0, sigma0, root0, bias0, Wp1, bp1, g1, mu1, sigma1, root1, bias1):
    raise NotImplementedError("write your pallas kernel here")


# speedup vs baseline: 2.2948x; 2.2948x over previous
import functools

import jax
import jax.numpy as jnp
from jax import lax
from jax.experimental import pallas as pl
from jax.experimental.pallas import tpu as pltpu
from jax.experimental.pallas import tpu_sc as plsc

N = 10000
E = 320000
EPS = 1e-15


_ROWS_BLK = 400
_EDGE_BLK = 512


def _prep_body(x_ref, g0_ref, root0_ref, z0_ref, r0_ref):
    xb = x_ref[...]
    z0_ref[...] = jnp.dot(xb, g0_ref[...], preferred_element_type=jnp.float32)
    r0_ref[...] = jnp.dot(xb, root0_ref[...], preferred_element_type=jnp.float32)


def _tc_prep(x, g0, root0):
    nblk = N // _ROWS_BLK
    return pl.pallas_call(
        _prep_body,
        grid=(nblk,),
        in_specs=[
            pl.BlockSpec((_ROWS_BLK, 128), lambda i: (i, 0)),
            pl.BlockSpec((128, 128), lambda i: (0, 0)),
            pl.BlockSpec((128, 64), lambda i: (0, 0)),
        ],
        out_specs=[
            pl.BlockSpec((_ROWS_BLK, 128), lambda i: (i, 0)),
            pl.BlockSpec((_ROWS_BLK, 64), lambda i: (i, 0)),
        ],
        out_shape=[
            jax.ShapeDtypeStruct((N, 128), jnp.float32),
            jax.ShapeDtypeStruct((N, 64), jnp.float32),
        ],
    )(x, g0, root0)


def _wts_body(ea_ref, wp0_ref, bp0_ref, mu0_ref, s0_ref,
              wp1_ref, bp1_ref, mu1_ref, s1_ref, w_ref):
    u = ea_ref[...]

    def layer(wp_ref, bp_ref, mu_ref, s_ref):
        p = jnp.tanh(jnp.dot(wp_ref[...], u,
                             preferred_element_type=jnp.float32)
                     + bp_ref[...])
        mu = mu_ref[...][:, :, None]
        inv = 1.0 / (EPS + s_ref[...][:, :, None] ** 2)
        diff = p[None, :, :] - mu
        return jnp.exp(jnp.sum(-0.5 * diff * diff * inv, axis=1))

    g0 = layer(wp0_ref, bp0_ref, mu0_ref, s0_ref)
    g1 = layer(wp1_ref, bp1_ref, mu1_ref, s1_ref)
    w_ref[...] = jnp.concatenate([g0, g1], axis=0)


def _tc_weights(ea_t, Wp0, bp0, mu0, sigma0, Wp1, bp1, mu1, sigma1):
    nblk = E // _EDGE_BLK
    small = pl.BlockSpec((2, 2), lambda i: (0, 0))
    smallb = pl.BlockSpec((2, 1), lambda i: (0, 0))
    return pl.pallas_call(
        _wts_body,
        grid=(nblk,),
        in_specs=[
            pl.BlockSpec((2, _EDGE_BLK), lambda i: (0, i)),
            small, smallb, small, small,
            small, smallb, small, small,
        ],
        out_specs=pl.BlockSpec((4, _EDGE_BLK), lambda i: (0, i)),
        out_shape=jax.ShapeDtypeStruct((4, E), jnp.float32),
    )(ea_t, Wp0, bp0, mu0, sigma0, Wp1, bp1, mu1, sigma1)


def _combine_body(pa_ref, pb_ref, r0_ref, b0_ref, g1e_ref, rt1_ref,
                  z1_ref, r1e_ref):
    s = pa_ref[...] + pb_ref[...]
    cnt = s[:, 64:65]
    inv = 1.0 / jnp.maximum(cnt, 1.0)
    h = s[:, :64] * inv + r0_ref[...] + b0_ref[...]
    h = jnp.where(h > 0, h, jnp.exp(h) - 1.0)
    z1_ref[...] = jnp.dot(h, g1e_ref[...], preferred_element_type=jnp.float32)
    r1 = jnp.dot(h, rt1_ref[...], preferred_element_type=jnp.float32)
    is40 = lax.broadcasted_iota(jnp.int32, (1, 48), 1) == 40
    r1e_ref[...] = r1 + jnp.where(is40, cnt, 0.0)


def _tc_combine(pa, pb, r0, bias0, g1e, root1p):
    nblk = N // _ROWS_BLK
    return pl.pallas_call(
        _combine_body,
        grid=(nblk,),
        in_specs=[
            pl.BlockSpec((_ROWS_BLK, 128), lambda i: (i, 0)),
            pl.BlockSpec((_ROWS_BLK, 128), lambda i: (i, 0)),
            pl.BlockSpec((_ROWS_BLK, 64), lambda i: (i, 0)),
            pl.BlockSpec((1, 64), lambda i: (0, 0)),
            pl.BlockSpec((64, 128), lambda i: (0, 0)),
            pl.BlockSpec((64, 48), lambda i: (0, 0)),
        ],
        out_specs=[
            pl.BlockSpec((_ROWS_BLK, 128), lambda i: (i, 0)),
            pl.BlockSpec((_ROWS_BLK, 48), lambda i: (i, 0)),
        ],
        out_shape=[
            jax.ShapeDtypeStruct((N, 128), jnp.float32),
            jax.ShapeDtypeStruct((N, 48), jnp.float32),
        ],
    )(pa, pb, r0, bias0, g1e, root1p)


def _final_body(pa_ref, pb_ref, r1e_ref, b1_ref, out_ref):
    s = pa_ref[:, :48] + pb_ref[:, :48]
    r1e = r1e_ref[...]
    cnt = r1e[:, 40:41]
    inv = 1.0 / jnp.maximum(cnt, 1.0)
    v = s * inv + r1e + b1_ref[...]
    mask = lax.broadcasted_iota(jnp.int32, (1, 48), 1) < 40
    vm = jnp.where(mask, v, -jnp.inf)
    m = jnp.max(vm, axis=1, keepdims=True)
    ex = jnp.where(mask, jnp.exp(v - m), 0.0)
    lse = jnp.log(jnp.sum(ex, axis=1, keepdims=True))
    out_ref[...] = (v - m - lse)[:, :40]


def _tc_final(pa, pb, r1e, bias1p):
    nblk = N // _ROWS_BLK
    return pl.pallas_call(
        _final_body,
        grid=(nblk,),
        in_specs=[
            pl.BlockSpec((_ROWS_BLK, 128), lambda i: (i, 0)),
            pl.BlockSpec((_ROWS_BLK, 128), lambda i: (i, 0)),
            pl.BlockSpec((_ROWS_BLK, 48), lambda i: (i, 0)),
            pl.BlockSpec((1, 48), lambda i: (0, 0)),
        ],
        out_specs=pl.BlockSpec((_ROWS_BLK, 40), lambda i: (i, 0)),
        out_shape=jax.ShapeDtypeStruct((N, 40), jnp.float32),
    )(pa, pb, r1e, bias1p)


def _make_sc_layer(f_in, f_half, f_msg, n_chunks, ones_col, blk):
    info = plsc.get_sparse_core_info()
    nc, ns = info.num_cores, info.num_subcores
    nw = nc * ns
    edges_per_w = E // nw
    nblk = edges_per_w // blk
    stripe = (N // ns) // 8 * 8
    rem = N - ns * stripe
    rem_base = ns * stripe
    msg_chunks = f_msg // 16

    mesh = plsc.VectorSubcoreMesh(core_axis_name="c", subcore_axis_name="s")

    @functools.partial(
        pl.kernel,
        mesh=mesh,
        out_type=pltpu.HBM((nc, N, f_msg), jnp.float32),
        scratch_types=[
            pltpu.VMEM((blk,), jnp.int32),
            pltpu.VMEM((blk,), jnp.int32),
            pltpu.VMEM((blk,), jnp.float32),
            pltpu.VMEM((blk,), jnp.float32),
            pltpu.VMEM((blk, f_in), jnp.float32),
            pltpu.VMEM((blk, f_msg), jnp.float32),
            pltpu.VMEM_SHARED((N, f_msg), jnp.float32),
            pltpu.SemaphoreType.DMA,
        ],
    )
    def sc_layer(z_hbm, src_hbm, dst_hbm, w0_hbm, w1_hbm, out_hbm,
                 src_v, dst_v, w0_v, w1_v, rows_v, msg_v, acc, sem):
        c = lax.axis_index("c")
        s = lax.axis_index("s")
        wid = s * nc + c
        base0 = wid * edges_per_w

        zeros16 = jnp.zeros((16,), jnp.float32)
        onehot = jnp.where(lax.iota(jnp.int32, 16) == 0, 1.0, 0.0)

        def zero_row(e, _):
            for f in range(msg_chunks):
                msg_v[e, pl.ds(16 * f, 16)] = zeros16
            return 0
        lax.fori_loop(0, blk, zero_row, 0)

        row0 = s * stripe
        done = 0
        while done < stripe:
            step = min(blk, stripe - done)
            pltpu.sync_copy(msg_v.at[pl.ds(0, step)],
                            acc.at[pl.ds(row0 + done, step)])
            done += step

        @pl.when(s == 0)
        def _zero_rem():
            pltpu.sync_copy(msg_v.at[pl.ds(0, rem)],
                            acc.at[pl.ds(rem_base, rem)])

        if ones_col >= 0:
            def ones_row(e, _):
                msg_v[e, pl.ds(16 * ones_col, 16)] = onehot
                return 0
            lax.fori_loop(0, blk, ones_row, 0)

        plsc.subcore_barrier()

        def do_block(b, _):
            base = pl.multiple_of(base0 + b * blk, 8)
            pltpu.sync_copy(src_hbm.at[pl.ds(base, blk)], src_v)
            pltpu.sync_copy(dst_hbm.at[pl.ds(base, blk)], dst_v)
            pltpu.sync_copy(w0_hbm.at[pl.ds(base, blk)], w0_v)
            pltpu.sync_copy(w1_hbm.at[pl.ds(base, blk)], w1_v)
            pltpu.async_copy(z_hbm.at[src_v], rows_v, sem).wait()

            def group(g, _):
                w0vec = w0_v[pl.ds(g * 16, 16)]
                w1vec = w1_v[pl.ds(g * 16, 16)]
                for j in range(16):
                    e = g * 16 + j
                    w0 = w0vec[j]
                    w1 = w1vec[j]
                    for f in range(n_chunks):
                        a = rows_v[e, pl.ds(16 * f, 16)]
                        bb = rows_v[e, pl.ds(f_half + 16 * f, 16)]
                        msg_v[e, pl.ds(16 * f, 16)] = w0 * a + w1 * bb
                return 0
            lax.fori_loop(0, blk // 16, group, 0)

            pltpu.sync_copy(msg_v, acc.at[dst_v], add=True)
            return 0
        lax.fori_loop(0, nblk, do_block, 0)

        plsc.subcore_barrier()
        pltpu.sync_copy(acc.at[pl.ds(row0, stripe)],
                        out_hbm.at[c, pl.ds(row0, stripe)])

        @pl.when(s == 0)
        def _copy_rem():
            pltpu.sync_copy(acc.at[pl.ds(rem_base, rem)],
                            out_hbm.at[c, pl.ds(rem_base, rem)])

    return sc_layer


def kernel(x, edge_index, edge_attr, Wp0, bp0, g0, mu0, sigma0, root0, bias0,
           Wp1, bp1, g1, mu1, sigma1, root1, bias1):
    ea_t = edge_attr.T

    z0, r0 = _tc_prep(x, g0, root0)
    wts = _tc_weights(ea_t, Wp0, bp0[:, None], mu0, sigma0,
                      Wp1, bp1[:, None], mu1, sigma1)

    src = edge_index[0]
    dst = edge_index[1]
    w00, w01, w10, w11 = wts[0], wts[1], wts[2], wts[3]

    sc0 = _make_sc_layer(f_in=128, f_half=64, f_msg=128, n_chunks=4,
                         ones_col=4, blk=80)
    part0 = sc0(z0, src, dst, w00, w01)

    g1e = jnp.concatenate(
        [g1[:, :40], jnp.zeros((64, 24), jnp.float32),
         g1[:, 40:], jnp.zeros((64, 24), jnp.float32)], axis=1)
    root1p = jnp.pad(root1, ((0, 0), (0, 8)))
    z1, r1e = _tc_combine(part0[0], part0[1], r0, bias0[None, :], g1e, root1p)

    sc1 = _make_sc_layer(f_in=128, f_half=64, f_msg=128, n_chunks=3,
                         ones_col=-1, blk=80)
    part1 = sc1(z1, src, dst, w10, w11)

    bias1p = jnp.pad(bias1, (0, 8))[None, :]
    return _tc_final(part1[0], part1[1], r1e, bias1p)


# speedup vs baseline: 4.1887x; 1.8253x over previous
import functools

import jax
import jax.numpy as jnp
from jax import lax
from jax.experimental import pallas as pl
from jax.experimental.pallas import tpu as pltpu
from jax.experimental.pallas import tpu_sc as plsc

N = 10000
E = 320000
EPS = 1e-15


_ROWS_BLK = 400
_EDGE_BLK = 512


def _prep_body(x_ref, g0_ref, root0_ref, z0_ref, r0_ref):
    xb = x_ref[...]
    z0_ref[...] = jnp.dot(xb, g0_ref[...], preferred_element_type=jnp.float32)
    r0_ref[...] = jnp.dot(xb, root0_ref[...], preferred_element_type=jnp.float32)


def _tc_prep(x, g0, root0):
    nblk = N // _ROWS_BLK
    return pl.pallas_call(
        _prep_body,
        grid=(nblk,),
        in_specs=[
            pl.BlockSpec((_ROWS_BLK, 128), lambda i: (i, 0)),
            pl.BlockSpec((128, 128), lambda i: (0, 0)),
            pl.BlockSpec((128, 64), lambda i: (0, 0)),
        ],
        out_specs=[
            pl.BlockSpec((_ROWS_BLK, 128), lambda i: (i, 0)),
            pl.BlockSpec((_ROWS_BLK, 64), lambda i: (i, 0)),
        ],
        out_shape=[
            jax.ShapeDtypeStruct((N, 128), jnp.float32),
            jax.ShapeDtypeStruct((N, 64), jnp.float32),
        ],
    )(x, g0, root0)


def _wts_body(ea_ref, wp0_ref, bp0_ref, mu0_ref, s0_ref,
              wp1_ref, bp1_ref, mu1_ref, s1_ref, w_ref):
    u = ea_ref[...]

    def layer(wp_ref, bp_ref, mu_ref, s_ref):
        p = jnp.tanh(jnp.dot(wp_ref[...], u,
                             preferred_element_type=jnp.float32)
                     + bp_ref[...])
        mu = mu_ref[...][:, :, None]
        inv = 1.0 / (EPS + s_ref[...][:, :, None] ** 2)
        diff = p[None, :, :] - mu
        return jnp.exp(jnp.sum(-0.5 * diff * diff * inv, axis=1))

    g0 = layer(wp0_ref, bp0_ref, mu0_ref, s0_ref)
    g1 = layer(wp1_ref, bp1_ref, mu1_ref, s1_ref)
    w_ref[...] = jnp.concatenate([g0, g1], axis=0)


def _tc_weights(ea_t, Wp0, bp0, mu0, sigma0, Wp1, bp1, mu1, sigma1):
    nblk = E // _EDGE_BLK
    small = pl.BlockSpec((2, 2), lambda i: (0, 0))
    smallb = pl.BlockSpec((2, 1), lambda i: (0, 0))
    return pl.pallas_call(
        _wts_body,
        grid=(nblk,),
        in_specs=[
            pl.BlockSpec((2, _EDGE_BLK), lambda i: (0, i)),
            small, smallb, small, small,
            small, smallb, small, small,
        ],
        out_specs=pl.BlockSpec((4, _EDGE_BLK), lambda i: (0, i)),
        out_shape=jax.ShapeDtypeStruct((4, E), jnp.float32),
    )(ea_t, Wp0, bp0, mu0, sigma0, Wp1, bp1, mu1, sigma1)


def _combine_body(pa_ref, pb_ref, r0_ref, b0_ref, g1e_ref, rt1_ref,
                  z1_ref, r1e_ref):
    s = pa_ref[...] + pb_ref[...]
    cnt = s[:, 64:65]
    inv = 1.0 / jnp.maximum(cnt, 1.0)
    h = s[:, :64] * inv + r0_ref[...] + b0_ref[...]
    h = jnp.where(h > 0, h, jnp.exp(h) - 1.0)
    z1_ref[...] = jnp.dot(h, g1e_ref[...], preferred_element_type=jnp.float32)
    r1 = jnp.dot(h, rt1_ref[...], preferred_element_type=jnp.float32)
    is40 = lax.broadcasted_iota(jnp.int32, (1, 48), 1) == 40
    r1e_ref[...] = r1 + jnp.where(is40, cnt, 0.0)


def _tc_combine(pa, pb, r0, bias0, g1e, root1p):
    nblk = N // _ROWS_BLK
    return pl.pallas_call(
        _combine_body,
        grid=(nblk,),
        in_specs=[
            pl.BlockSpec((_ROWS_BLK, 128), lambda i: (i, 0)),
            pl.BlockSpec((_ROWS_BLK, 128), lambda i: (i, 0)),
            pl.BlockSpec((_ROWS_BLK, 64), lambda i: (i, 0)),
            pl.BlockSpec((1, 64), lambda i: (0, 0)),
            pl.BlockSpec((64, 128), lambda i: (0, 0)),
            pl.BlockSpec((64, 48), lambda i: (0, 0)),
        ],
        out_specs=[
            pl.BlockSpec((_ROWS_BLK, 128), lambda i: (i, 0)),
            pl.BlockSpec((_ROWS_BLK, 48), lambda i: (i, 0)),
        ],
        out_shape=[
            jax.ShapeDtypeStruct((N, 128), jnp.float32),
            jax.ShapeDtypeStruct((N, 48), jnp.float32),
        ],
    )(pa, pb, r0, bias0, g1e, root1p)


def _final_body(pa_ref, pb_ref, r1e_ref, b1_ref, out_ref):
    s = pa_ref[:, :48] + pb_ref[:, :48]
    r1e = r1e_ref[...]
    cnt = r1e[:, 40:41]
    inv = 1.0 / jnp.maximum(cnt, 1.0)
    v = s * inv + r1e + b1_ref[...]
    mask = lax.broadcasted_iota(jnp.int32, (1, 48), 1) < 40
    vm = jnp.where(mask, v, -jnp.inf)
    m = jnp.max(vm, axis=1, keepdims=True)
    ex = jnp.where(mask, jnp.exp(v - m), 0.0)
    lse = jnp.log(jnp.sum(ex, axis=1, keepdims=True))
    out_ref[...] = (v - m - lse)[:, :40]


def _tc_final(pa, pb, r1e, bias1p):
    nblk = N // _ROWS_BLK
    return pl.pallas_call(
        _final_body,
        grid=(nblk,),
        in_specs=[
            pl.BlockSpec((_ROWS_BLK, 128), lambda i: (i, 0)),
            pl.BlockSpec((_ROWS_BLK, 128), lambda i: (i, 0)),
            pl.BlockSpec((_ROWS_BLK, 48), lambda i: (i, 0)),
            pl.BlockSpec((1, 48), lambda i: (0, 0)),
        ],
        out_specs=pl.BlockSpec((_ROWS_BLK, 40), lambda i: (i, 0)),
        out_shape=jax.ShapeDtypeStruct((N, 40), jnp.float32),
    )(pa, pb, r1e, bias1p)


def _make_sc_layer(f_half, n_chunks, ones_col, w_row):
    f_in = 128
    f_msg = 128
    blk = 80
    chunk = 2000
    info = plsc.get_sparse_core_info()
    nc, ns = info.num_cores, info.num_subcores
    nw = nc * ns
    edges_per_w = E // nw
    nchunks_e = edges_per_w // chunk
    nblk = chunk // blk
    stripe = (N // ns) // 8 * 8
    rem = N - ns * stripe
    rem_base = ns * stripe
    msg_chunks = f_msg // 16

    mesh = plsc.VectorSubcoreMesh(core_axis_name="c", subcore_axis_name="s")

    @functools.partial(
        pl.kernel,
        mesh=mesh,
        out_type=pltpu.HBM((nc, N, f_msg), jnp.float32),
        scratch_types=[
            pltpu.VMEM((chunk,), jnp.int32),
            pltpu.VMEM((chunk,), jnp.int32),
            pltpu.VMEM((chunk,), jnp.float32),
            pltpu.VMEM((chunk,), jnp.float32),
            pltpu.VMEM((blk, f_in), jnp.float32),
            pltpu.VMEM((blk, f_in), jnp.float32),
            pltpu.VMEM((blk, f_msg), jnp.float32),
            pltpu.VMEM((blk, f_msg), jnp.float32),
            pltpu.VMEM_SHARED((N, f_msg), jnp.float32),
            pltpu.SemaphoreType.DMA,
            pltpu.SemaphoreType.DMA,
            pltpu.SemaphoreType.DMA,
            pltpu.SemaphoreType.DMA,
        ],
    )
    def sc_layer(z_hbm, ei_hbm, w_hbm, out_hbm,
                 src_c, dst_c, w0_c, w1_c, rows_a, rows_b, msg_a, msg_b,
                 acc, sem_ga, sem_gb, sem_sa, sem_sb):
        c = lax.axis_index("c")
        s = lax.axis_index("s")
        wid = s * nc + c
        base0 = wid * edges_per_w

        zeros16 = jnp.zeros((16,), jnp.float32)
        onehot = jnp.where(lax.iota(jnp.int32, 16) == 0, 1.0, 0.0)

        def zero_row(e, _):
            for f in range(msg_chunks):
                msg_a[e, pl.ds(16 * f, 16)] = zeros16
                msg_b[e, pl.ds(16 * f, 16)] = zeros16
            return 0
        lax.fori_loop(0, blk, zero_row, 0)

        row0 = s * stripe
        done = 0
        while done < stripe:
            step = min(blk, stripe - done)
            pltpu.sync_copy(msg_a.at[pl.ds(0, step)],
                            acc.at[pl.ds(row0 + done, step)])
            done += step

        @pl.when(s == 0)
        def _zero_rem():
            pltpu.sync_copy(msg_a.at[pl.ds(0, rem)],
                            acc.at[pl.ds(rem_base, rem)])

        if ones_col >= 0:
            def ones_row(e, _):
                msg_a[e, pl.ds(16 * ones_col, 16)] = onehot
                msg_b[e, pl.ds(16 * ones_col, 16)] = onehot
                return 0
            lax.fori_loop(0, blk, ones_row, 0)

        plsc.subcore_barrier()

        rows_bufs = (rows_a, rows_b)
        msg_bufs = (msg_a, msg_b)
        gsems = (sem_ga, sem_gb)
        ssems = (sem_sa, sem_sb)

        def issue_gather(bi, p):
            pltpu.async_copy(z_hbm.at[src_c.at[pl.ds(bi * blk, blk)]],
                             rows_bufs[p], gsems[p])

        def wait_gather(p):
            pltpu.make_async_copy(z_hbm.at[pl.ds(0, blk)],
                                  rows_bufs[p], gsems[p]).wait()

        def issue_scatter(bi, p):
            pltpu.async_copy(msg_bufs[p],
                             acc.at[dst_c.at[pl.ds(bi * blk, blk)]],
                             ssems[p], add=True)

        def wait_scatter(p):
            pltpu.make_async_copy(msg_bufs[p], acc.at[pl.ds(0, blk)],
                                  ssems[p]).wait()

        def compute(bi, p):
            rows_v = rows_bufs[p]
            msg_v = msg_bufs[p]

            def group(g, _):
                off = bi * blk + g * 16
                w0vec = w0_c[pl.ds(off, 16)]
                w1vec = w1_c[pl.ds(off, 16)]
                for j in range(16):
                    e = g * 16 + j
                    w0 = w0vec[j]
                    w1 = w1vec[j]
                    for f in range(n_chunks):
                        a = rows_v[e, pl.ds(16 * f, 16)]
                        bb = rows_v[e, pl.ds(f_half + 16 * f, 16)]
                        msg_v[e, pl.ds(16 * f, 16)] = w0 * a + w1 * bb
                return 0
            lax.fori_loop(0, blk // 16, group, 0)

        def do_chunk(ci, _):
            base = pl.multiple_of(base0 + ci * chunk, 8)
            pltpu.sync_copy(ei_hbm.at[pl.ds(base, chunk)], src_c)
            pltpu.sync_copy(ei_hbm.at[pl.ds(E + base, chunk)], dst_c)
            pltpu.sync_copy(w_hbm.at[pl.ds(w_row * E + base, chunk)], w0_c)
            pltpu.sync_copy(w_hbm.at[pl.ds((w_row + 1) * E + base, chunk)],
                            w1_c)

            issue_gather(0, 0)
            issue_gather(1, 1)

            def body(b, _):
                for p in range(2):
                    @pl.when(b % 2 == p)
                    def _run():
                        wait_gather(p)

                        @pl.when(b >= 2)
                        def _ws():
                            wait_scatter(p)
                        compute(b, p)
                        issue_scatter(b, p)

                        @pl.when(b + 2 < nblk)
                        def _ig():
                            issue_gather(b + 2, p)
                return 0
            lax.fori_loop(0, nblk, body, 0)

            wait_scatter(0)
            wait_scatter(1)
            return 0
        lax.fori_loop(0, nchunks_e, do_chunk, 0)

        plsc.subcore_barrier()
        pltpu.sync_copy(acc.at[pl.ds(row0, stripe)],
                        out_hbm.at[c, pl.ds(row0, stripe)])

        @pl.when(s == 0)
        def _copy_rem():
            pltpu.sync_copy(acc.at[pl.ds(rem_base, rem)],
                            out_hbm.at[c, pl.ds(rem_base, rem)])

    return sc_layer


def kernel(x, edge_index, edge_attr, Wp0, bp0, g0, mu0, sigma0, root0, bias0,
           Wp1, bp1, g1, mu1, sigma1, root1, bias1):
    ea_t = edge_attr.T

    z0, r0 = _tc_prep(x, g0, root0)
    wts = _tc_weights(ea_t, Wp0, bp0[:, None], mu0, sigma0,
                      Wp1, bp1[:, None], mu1, sigma1)

    ei_flat = edge_index.reshape(2 * E)
    w_flat = wts.reshape(4 * E)

    sc0 = _make_sc_layer(f_half=64, n_chunks=4, ones_col=4, w_row=0)
    part0 = sc0(z0, ei_flat, w_flat)

    g1e = jnp.concatenate(
        [g1[:, :40], jnp.zeros((64, 24), jnp.float32),
         g1[:, 40:], jnp.zeros((64, 24), jnp.float32)], axis=1)
    root1p = jnp.pad(root1, ((0, 0), (0, 8)))
    z1, r1e = _tc_combine(part0[0], part0[1], r0, bias0[None, :], g1e, root1p)

    sc1 = _make_sc_layer(f_half=64, n_chunks=3, ones_col=-1, w_row=2)
    part1 = sc1(z1, ei_flat, w_flat)

    bias1p = jnp.pad(bias1, (0, 8))[None, :]
    return _tc_final(part1[0], part1[1], r1e, bias1p)


# speedup vs baseline: 6.2190x; 1.4847x over previous
import functools

import jax
import jax.numpy as jnp
from jax import lax
from jax.experimental import pallas as pl
from jax.experimental.pallas import tpu as pltpu
from jax.experimental.pallas import tpu_sc as plsc

N = 10000
E = 320000
EPS = 1e-15


_ROWS_BLK = 1000
_EDGE_BLK = 12800


def _prep_body(x_ref, g0_ref, root0_ref, z0_ref, r0_ref):
    xb = x_ref[...]
    z0_ref[...] = jnp.dot(xb, g0_ref[...], preferred_element_type=jnp.float32)
    r0_ref[...] = jnp.dot(xb, root0_ref[...], preferred_element_type=jnp.float32)


def _tc_prep(x, g0, root0):
    nblk = N // _ROWS_BLK
    return pl.pallas_call(
        _prep_body,
        grid=(nblk,),
        in_specs=[
            pl.BlockSpec((_ROWS_BLK, 128), lambda i: (i, 0)),
            pl.BlockSpec((128, 128), lambda i: (0, 0)),
            pl.BlockSpec((128, 64), lambda i: (0, 0)),
        ],
        out_specs=[
            pl.BlockSpec((_ROWS_BLK, 128), lambda i: (i, 0)),
            pl.BlockSpec((_ROWS_BLK, 64), lambda i: (i, 0)),
        ],
        out_shape=[
            jax.ShapeDtypeStruct((N, 128), jnp.float32),
            jax.ShapeDtypeStruct((N, 64), jnp.float32),
        ],
    )(x, g0, root0)


def _wts_body(ea_ref, wp0_ref, bp0_ref, mu0_ref, s0_ref,
              wp1_ref, bp1_ref, mu1_ref, s1_ref, w_ref):
    u = ea_ref[...]

    def layer(wp_ref, bp_ref, mu_ref, s_ref):
        p = jnp.tanh(jnp.dot(wp_ref[...], u,
                             preferred_element_type=jnp.float32)
                     + bp_ref[...])
        mu = mu_ref[...][:, :, None]
        inv = 1.0 / (EPS + s_ref[...][:, :, None] ** 2)
        diff = p[None, :, :] - mu
        return jnp.exp(jnp.sum(-0.5 * diff * diff * inv, axis=1))

    g0 = layer(wp0_ref, bp0_ref, mu0_ref, s0_ref)
    g1 = layer(wp1_ref, bp1_ref, mu1_ref, s1_ref)
    w_ref[...] = jnp.concatenate([g0, g1], axis=0)


def _tc_weights(ea_t, Wp0, bp0, mu0, sigma0, Wp1, bp1, mu1, sigma1):
    nblk = E // _EDGE_BLK
    small = pl.BlockSpec((2, 2), lambda i: (0, 0))
    smallb = pl.BlockSpec((2, 1), lambda i: (0, 0))
    return pl.pallas_call(
        _wts_body,
        grid=(nblk,),
        in_specs=[
            pl.BlockSpec((2, _EDGE_BLK), lambda i: (0, i)),
            small, smallb, small, small,
            small, smallb, small, small,
        ],
        out_specs=pl.BlockSpec((4, _EDGE_BLK), lambda i: (0, i)),
        out_shape=jax.ShapeDtypeStruct((4, E), jnp.float32),
    )(ea_t, Wp0, bp0, mu0, sigma0, Wp1, bp1, mu1, sigma1)


def _combine_body(pa_ref, pb_ref, r0_ref, b0_ref, g1e_ref, rt1_ref,
                  z1_ref, r1e_ref):
    s = pa_ref[...] + pb_ref[...]
    cnt = s[:, 64:65]
    inv = 1.0 / jnp.maximum(cnt, 1.0)
    h = s[:, :64] * inv + r0_ref[...] + b0_ref[...]
    h = jnp.where(h > 0, h, jnp.exp(h) - 1.0)
    z1_ref[...] = jnp.dot(h, g1e_ref[...], preferred_element_type=jnp.float32)
    r1 = jnp.dot(h, rt1_ref[...], preferred_element_type=jnp.float32)
    is40 = lax.broadcasted_iota(jnp.int32, (1, 48), 1) == 40
    r1e_ref[...] = r1 + jnp.where(is40, cnt, 0.0)


def _tc_combine(pa, pb, r0, bias0, g1e, root1p):
    nblk = N // _ROWS_BLK
    return pl.pallas_call(
        _combine_body,
        grid=(nblk,),
        in_specs=[
            pl.BlockSpec((_ROWS_BLK, 128), lambda i: (i, 0)),
            pl.BlockSpec((_ROWS_BLK, 128), lambda i: (i, 0)),
            pl.BlockSpec((_ROWS_BLK, 64), lambda i: (i, 0)),
            pl.BlockSpec((1, 64), lambda i: (0, 0)),
            pl.BlockSpec((64, 128), lambda i: (0, 0)),
            pl.BlockSpec((64, 48), lambda i: (0, 0)),
        ],
        out_specs=[
            pl.BlockSpec((_ROWS_BLK, 128), lambda i: (i, 0)),
            pl.BlockSpec((_ROWS_BLK, 48), lambda i: (i, 0)),
        ],
        out_shape=[
            jax.ShapeDtypeStruct((N, 128), jnp.float32),
            jax.ShapeDtypeStruct((N, 48), jnp.float32),
        ],
    )(pa, pb, r0, bias0, g1e, root1p)


def _final_body(pa_ref, pb_ref, r1e_ref, b1_ref, out_ref):
    s = pa_ref[:, :48] + pb_ref[:, :48]
    r1e = r1e_ref[...]
    cnt = r1e[:, 40:41]
    inv = 1.0 / jnp.maximum(cnt, 1.0)
    v = s * inv + r1e + b1_ref[...]
    mask = lax.broadcasted_iota(jnp.int32, (1, 48), 1) < 40
    vm = jnp.where(mask, v, -jnp.inf)
    m = jnp.max(vm, axis=1, keepdims=True)
    ex = jnp.where(mask, jnp.exp(v - m), 0.0)
    lse = jnp.log(jnp.sum(ex, axis=1, keepdims=True))
    out_ref[...] = (v - m - lse)[:, :40]


def _tc_final(pa, pb, r1e, bias1p):
    nblk = N // _ROWS_BLK
    return pl.pallas_call(
        _final_body,
        grid=(nblk,),
        in_specs=[
            pl.BlockSpec((_ROWS_BLK, 128), lambda i: (i, 0)),
            pl.BlockSpec((_ROWS_BLK, 128), lambda i: (i, 0)),
            pl.BlockSpec((_ROWS_BLK, 48), lambda i: (i, 0)),
            pl.BlockSpec((1, 48), lambda i: (0, 0)),
        ],
        out_specs=pl.BlockSpec((_ROWS_BLK, 40), lambda i: (i, 0)),
        out_shape=jax.ShapeDtypeStruct((N, 40), jnp.float32),
    )(pa, pb, r1e, bias1p)


def _make_sc_layer(f_half, n_chunks, ones_col, w_row):
    f_in = 128
    f_msg = 128
    blk = 80
    chunk = 2000
    info = plsc.get_sparse_core_info()
    nc, ns = info.num_cores, info.num_subcores
    nw = nc * ns
    edges_per_w = E // nw
    nchunks_e = edges_per_w // chunk
    nblk = chunk // blk
    stripe = (N // ns) // 8 * 8
    rem = N - ns * stripe
    rem_base = ns * stripe
    msg_chunks = f_msg // 16

    mesh = plsc.VectorSubcoreMesh(core_axis_name="c", subcore_axis_name="s")

    @functools.partial(
        pl.kernel,
        mesh=mesh,
        out_type=pltpu.HBM((nc, N, f_msg), jnp.float32),
        scratch_types=[
            pltpu.VMEM((chunk,), jnp.int32),
            pltpu.VMEM((chunk,), jnp.int32),
            pltpu.VMEM((chunk,), jnp.float32),
            pltpu.VMEM((chunk,), jnp.float32),
            pltpu.VMEM((blk, f_in), jnp.float32),
            pltpu.VMEM((blk, f_in), jnp.float32),
            pltpu.VMEM((blk, f_msg), jnp.float32),
            pltpu.VMEM((blk, f_msg), jnp.float32),
            pltpu.VMEM_SHARED((N, f_msg), jnp.float32),
            pltpu.SemaphoreType.DMA,
            pltpu.SemaphoreType.DMA,
            pltpu.SemaphoreType.DMA,
            pltpu.SemaphoreType.DMA,
        ],
    )
    def sc_layer(z_hbm, ei_hbm, w_hbm, out_hbm,
                 src_c, dst_c, w0_c, w1_c, rows_a, rows_b, msg_a, msg_b,
                 acc, sem_ga, sem_gb, sem_sa, sem_sb):
        c = lax.axis_index("c")
        s = lax.axis_index("s")
        wid = s * nc + c
        base0 = wid * edges_per_w

        zeros16 = jnp.zeros((16,), jnp.float32)
        onehot = jnp.where(lax.iota(jnp.int32, 16) == 0, 1.0, 0.0)

        def zero_row(e, _):
            for f in range(msg_chunks):
                msg_a[e, pl.ds(16 * f, 16)] = zeros16
                msg_b[e, pl.ds(16 * f, 16)] = zeros16
            return 0
        lax.fori_loop(0, blk, zero_row, 0)

        row0 = s * stripe
        done = 0
        while done < stripe:
            step = min(blk, stripe - done)
            pltpu.sync_copy(msg_a.at[pl.ds(0, step)],
                            acc.at[pl.ds(row0 + done, step)])
            done += step

        @pl.when(s == 0)
        def _zero_rem():
            pltpu.sync_copy(msg_a.at[pl.ds(0, rem)],
                            acc.at[pl.ds(rem_base, rem)])

        if ones_col >= 0:
            def ones_row(e, _):
                msg_a[e, pl.ds(16 * ones_col, 16)] = onehot
                msg_b[e, pl.ds(16 * ones_col, 16)] = onehot
                return 0
            lax.fori_loop(0, blk, ones_row, 0)

        plsc.subcore_barrier()

        rows_bufs = (rows_a, rows_b)
        msg_bufs = (msg_a, msg_b)
        gsems = (sem_ga, sem_gb)
        ssems = (sem_sa, sem_sb)

        def issue_gather(bi, p):
            pltpu.async_copy(z_hbm.at[src_c.at[pl.ds(bi * blk, blk)]],
                             rows_bufs[p], gsems[p])

        def wait_gather(p):
            pltpu.make_async_copy(z_hbm.at[pl.ds(0, blk)],
                                  rows_bufs[p], gsems[p]).wait()

        def issue_scatter(bi, p):
            pltpu.async_copy(msg_bufs[p],
                             acc.at[dst_c.at[pl.ds(bi * blk, blk)]],
                             ssems[p], add=True)

        def wait_scatter(p):
            pltpu.make_async_copy(msg_bufs[p], acc.at[pl.ds(0, blk)],
                                  ssems[p]).wait()

        def compute(bi, p):
            rows_v = rows_bufs[p]
            msg_v = msg_bufs[p]

            def group(g, _):
                off = bi * blk + g * 16
                w0vec = w0_c[pl.ds(off, 16)]
                w1vec = w1_c[pl.ds(off, 16)]
                for j in range(16):
                    e = g * 16 + j
                    w0 = w0vec[j]
                    w1 = w1vec[j]
                    for f in range(n_chunks):
                        a = rows_v[e, pl.ds(16 * f, 16)]
                        bb = rows_v[e, pl.ds(f_half + 16 * f, 16)]
                        msg_v[e, pl.ds(16 * f, 16)] = w0 * a + w1 * bb
                return 0
            lax.fori_loop(0, blk // 16, group, 0)

        def do_chunk(ci, _):
            base = pl.multiple_of(base0 + ci * chunk, 8)
            pltpu.sync_copy(ei_hbm.at[pl.ds(base, chunk)], src_c)
            pltpu.sync_copy(ei_hbm.at[pl.ds(E + base, chunk)], dst_c)
            pltpu.sync_copy(w_hbm.at[pl.ds(w_row * E + base, chunk)], w0_c)
            pltpu.sync_copy(w_hbm.at[pl.ds((w_row + 1) * E + base, chunk)],
                            w1_c)

            issue_gather(0, 0)
            issue_gather(1, 1)

            def body(b, _):
                for p in range(2):
                    @pl.when(b % 2 == p)
                    def _run():
                        wait_gather(p)

                        @pl.when(b >= 2)
                        def _ws():
                            wait_scatter(p)
                        compute(b, p)
                        issue_scatter(b, p)

                        @pl.when(b + 2 < nblk)
                        def _ig():
                            issue_gather(b + 2, p)
                return 0
            lax.fori_loop(0, nblk, body, 0)

            wait_scatter(0)
            wait_scatter(1)
            return 0
        lax.fori_loop(0, nchunks_e, do_chunk, 0)

        plsc.subcore_barrier()
        pltpu.sync_copy(acc.at[pl.ds(row0, stripe)],
                        out_hbm.at[c, pl.ds(row0, stripe)])

        @pl.when(s == 0)
        def _copy_rem():
            pltpu.sync_copy(acc.at[pl.ds(rem_base, rem)],
                            out_hbm.at[c, pl.ds(rem_base, rem)])

    return sc_layer


def kernel(x, edge_index, edge_attr, Wp0, bp0, g0, mu0, sigma0, root0, bias0,
           Wp1, bp1, g1, mu1, sigma1, root1, bias1):
    ea_t = edge_attr.T

    z0, r0 = _tc_prep(x, g0, root0)
    wts = _tc_weights(ea_t, Wp0, bp0[:, None], mu0, sigma0,
                      Wp1, bp1[:, None], mu1, sigma1)

    ei_flat = edge_index.reshape(2 * E)
    w_flat = wts.reshape(4 * E)

    sc0 = _make_sc_layer(f_half=64, n_chunks=4, ones_col=4, w_row=0)
    part0 = sc0(z0, ei_flat, w_flat)

    g1e = jnp.concatenate(
        [g1[:, :40], jnp.zeros((64, 24), jnp.float32),
         g1[:, 40:], jnp.zeros((64, 24), jnp.float32)], axis=1)
    root1p = jnp.pad(root1, ((0, 0), (0, 8)))
    z1, r1e = _tc_combine(part0[0], part0[1], r0, bias0[None, :], g1e, root1p)

    sc1 = _make_sc_layer(f_half=64, n_chunks=3, ones_col=-1, w_row=2)
    part1 = sc1(z1, ei_flat, w_flat)

    bias1p = jnp.pad(bias1, (0, 8))[None, :]
    return _tc_final(part1[0], part1[1], r1e, bias1p)


# speedup vs baseline: 9.1086x; 1.4646x over previous
import functools

import jax
import jax.numpy as jnp
from jax import lax
from jax.experimental import pallas as pl
from jax.experimental.pallas import tpu as pltpu
from jax.experimental.pallas import tpu_sc as plsc

N = 10000
E = 320000
EPS = 1e-15


_ROWS_BLK = 1000
_EDGE_BLK = 12800


def _prep_body(x_ref, g0_ref, root0_ref, z0_ref, r0_ref):
    xb = x_ref[...]
    z0_ref[...] = jnp.dot(xb, g0_ref[...], preferred_element_type=jnp.float32)
    r0_ref[...] = jnp.dot(xb, root0_ref[...], preferred_element_type=jnp.float32)


def _tc_prep(x, g0, root0):
    nblk = N // _ROWS_BLK
    return pl.pallas_call(
        _prep_body,
        grid=(nblk,),
        in_specs=[
            pl.BlockSpec((_ROWS_BLK, 128), lambda i: (i, 0)),
            pl.BlockSpec((128, 128), lambda i: (0, 0)),
            pl.BlockSpec((128, 64), lambda i: (0, 0)),
        ],
        out_specs=[
            pl.BlockSpec((_ROWS_BLK, 128), lambda i: (i, 0)),
            pl.BlockSpec((_ROWS_BLK, 64), lambda i: (i, 0)),
        ],
        out_shape=[
            jax.ShapeDtypeStruct((N, 128), jnp.float32),
            jax.ShapeDtypeStruct((N, 64), jnp.float32),
        ],
    )(x, g0, root0)


def _wts_body(ea_ref, wp0_ref, bp0_ref, mu0_ref, s0_ref,
              wp1_ref, bp1_ref, mu1_ref, s1_ref, w_ref):
    u = ea_ref[...]

    def layer(wp_ref, bp_ref, mu_ref, s_ref):
        p = jnp.tanh(jnp.dot(wp_ref[...], u,
                             preferred_element_type=jnp.float32)
                     + bp_ref[...])
        mu = mu_ref[...][:, :, None]
        inv = 1.0 / (EPS + s_ref[...][:, :, None] ** 2)
        diff = p[None, :, :] - mu
        return jnp.exp(jnp.sum(-0.5 * diff * diff * inv, axis=1))

    g0 = layer(wp0_ref, bp0_ref, mu0_ref, s0_ref)
    g1 = layer(wp1_ref, bp1_ref, mu1_ref, s1_ref)
    w_ref[...] = jnp.concatenate([g0, g1], axis=0)


def _tc_weights(ea_t, Wp0, bp0, mu0, sigma0, Wp1, bp1, mu1, sigma1):
    nblk = E // _EDGE_BLK
    small = pl.BlockSpec((2, 2), lambda i: (0, 0))
    smallb = pl.BlockSpec((2, 1), lambda i: (0, 0))
    return pl.pallas_call(
        _wts_body,
        grid=(nblk,),
        in_specs=[
            pl.BlockSpec((2, _EDGE_BLK), lambda i: (0, i)),
            small, smallb, small, small,
            small, smallb, small, small,
        ],
        out_specs=pl.BlockSpec((4, _EDGE_BLK), lambda i: (0, i)),
        out_shape=jax.ShapeDtypeStruct((4, E), jnp.float32),
    )(ea_t, Wp0, bp0, mu0, sigma0, Wp1, bp1, mu1, sigma1)


def _combine_body(pa_ref, pb_ref, r0_ref, b0_ref, g1e_ref, rt1_ref,
                  z1_ref, r1e_ref):
    s = pa_ref[...] + pb_ref[...]
    cnt = s[:, 64:65]
    inv = 1.0 / jnp.maximum(cnt, 1.0)
    h = s[:, :64] * inv + r0_ref[...] + b0_ref[...]
    h = jnp.where(h > 0, h, jnp.exp(h) - 1.0)
    z1_ref[...] = jnp.dot(h, g1e_ref[...], preferred_element_type=jnp.float32)
    r1 = jnp.dot(h, rt1_ref[...], preferred_element_type=jnp.float32)
    is40 = lax.broadcasted_iota(jnp.int32, (1, 48), 1) == 40
    r1e_ref[...] = r1 + jnp.where(is40, cnt, 0.0)


def _tc_combine(pa, pb, r0, bias0, g1e, root1p):
    nblk = N // _ROWS_BLK
    return pl.pallas_call(
        _combine_body,
        grid=(nblk,),
        in_specs=[
            pl.BlockSpec((_ROWS_BLK, 128), lambda i: (i, 0)),
            pl.BlockSpec((_ROWS_BLK, 128), lambda i: (i, 0)),
            pl.BlockSpec((_ROWS_BLK, 64), lambda i: (i, 0)),
            pl.BlockSpec((1, 64), lambda i: (0, 0)),
            pl.BlockSpec((64, 128), lambda i: (0, 0)),
            pl.BlockSpec((64, 48), lambda i: (0, 0)),
        ],
        out_specs=[
            pl.BlockSpec((_ROWS_BLK, 128), lambda i: (i, 0)),
            pl.BlockSpec((_ROWS_BLK, 48), lambda i: (i, 0)),
        ],
        out_shape=[
            jax.ShapeDtypeStruct((N, 128), jnp.float32),
            jax.ShapeDtypeStruct((N, 48), jnp.float32),
        ],
    )(pa, pb, r0, bias0, g1e, root1p)


def _final_body(pa_ref, pb_ref, r1e_ref, b1_ref, out_ref):
    s = pa_ref[:, :48] + pb_ref[:, :48]
    r1e = r1e_ref[...]
    cnt = r1e[:, 40:41]
    inv = 1.0 / jnp.maximum(cnt, 1.0)
    v = s * inv + r1e + b1_ref[...]
    mask = lax.broadcasted_iota(jnp.int32, (1, 48), 1) < 40
    vm = jnp.where(mask, v, -jnp.inf)
    m = jnp.max(vm, axis=1, keepdims=True)
    ex = jnp.where(mask, jnp.exp(v - m), 0.0)
    lse = jnp.log(jnp.sum(ex, axis=1, keepdims=True))
    out_ref[...] = (v - m - lse)[:, :40]


def _tc_final(pa, pb, r1e, bias1p):
    nblk = N // _ROWS_BLK
    return pl.pallas_call(
        _final_body,
        grid=(nblk,),
        in_specs=[
            pl.BlockSpec((_ROWS_BLK, 128), lambda i: (i, 0)),
            pl.BlockSpec((_ROWS_BLK, 128), lambda i: (i, 0)),
            pl.BlockSpec((_ROWS_BLK, 48), lambda i: (i, 0)),
            pl.BlockSpec((1, 48), lambda i: (0, 0)),
        ],
        out_specs=pl.BlockSpec((_ROWS_BLK, 40), lambda i: (i, 0)),
        out_shape=jax.ShapeDtypeStruct((N, 40), jnp.float32),
    )(pa, pb, r1e, bias1p)


def _make_sc_layer(f_half, n_chunks, ones_col, w_row):
    f_in = 128
    f_msg = 128
    blk = 80
    chunk = 2000
    info = plsc.get_sparse_core_info()
    nc, ns = info.num_cores, info.num_subcores
    nw = nc * ns
    edges_per_w = E // nw
    nchunks_e = edges_per_w // chunk
    nblk = chunk // blk
    stripe = (N // ns) // 8 * 8
    rem = N - ns * stripe
    rem_base = ns * stripe
    msg_chunks = f_msg // 16

    mesh = plsc.VectorSubcoreMesh(core_axis_name="c", subcore_axis_name="s")

    @functools.partial(
        pl.kernel,
        mesh=mesh,
        out_type=pltpu.HBM((nc, N, f_msg), jnp.float32),
        scratch_types=[
            pltpu.VMEM((chunk,), jnp.int32),
            pltpu.VMEM((chunk,), jnp.int32),
            pltpu.VMEM((chunk,), jnp.float32),
            pltpu.VMEM((chunk,), jnp.float32),
            pltpu.VMEM((blk, f_in), jnp.float32),
            pltpu.VMEM((blk, f_in), jnp.float32),
            pltpu.VMEM((blk, f_msg), jnp.float32),
            pltpu.VMEM((blk, f_msg), jnp.float32),
            pltpu.VMEM_SHARED((N, f_msg), jnp.float32),
            pltpu.SemaphoreType.DMA,
            pltpu.SemaphoreType.DMA,
            pltpu.SemaphoreType.DMA,
            pltpu.SemaphoreType.DMA,
        ],
    )
    def sc_layer(z_hbm, ei_hbm, w_hbm, out_hbm,
                 src_c, dst_c, w0_c, w1_c, rows_a, rows_b, msg_a, msg_b,
                 acc, sem_ga, sem_gb, sem_sa, sem_sb):
        c = lax.axis_index("c")
        s = lax.axis_index("s")
        wid = s * nc + c
        base0 = wid * edges_per_w

        zeros16 = jnp.zeros((16,), jnp.float32)
        onehot = jnp.where(lax.iota(jnp.int32, 16) == 0, 1.0, 0.0)

        def zero_row(e, _):
            for f in range(msg_chunks):
                msg_a[e, pl.ds(16 * f, 16)] = zeros16
                msg_b[e, pl.ds(16 * f, 16)] = zeros16
            return 0
        lax.fori_loop(0, blk, zero_row, 0)

        row0 = s * stripe
        done = 0
        while done < stripe:
            step = min(blk, stripe - done)
            pltpu.sync_copy(msg_a.at[pl.ds(0, step)],
                            acc.at[pl.ds(row0 + done, step)])
            done += step

        @pl.when(s == 0)
        def _zero_rem():
            pltpu.sync_copy(msg_a.at[pl.ds(0, rem)],
                            acc.at[pl.ds(rem_base, rem)])

        if ones_col >= 0:
            def ones_row(e, _):
                msg_a[e, pl.ds(16 * ones_col, 16)] = onehot
                msg_b[e, pl.ds(16 * ones_col, 16)] = onehot
                return 0
            lax.fori_loop(0, blk, ones_row, 0)

        plsc.subcore_barrier()

        rows_bufs = (rows_a, rows_b)
        msg_bufs = (msg_a, msg_b)
        gsems = (sem_ga, sem_gb)
        ssems = (sem_sa, sem_sb)

        def issue_gather(bi, p):
            pltpu.async_copy(z_hbm.at[src_c.at[pl.ds(bi * blk, blk)]],
                             rows_bufs[p], gsems[p])

        def wait_gather(p):
            pltpu.make_async_copy(z_hbm.at[pl.ds(0, blk)],
                                  rows_bufs[p], gsems[p]).wait()

        def issue_scatter(bi, p):
            pltpu.async_copy(msg_bufs[p],
                             acc.at[dst_c.at[pl.ds(bi * blk, blk)]],
                             ssems[p], add=True)

        def wait_scatter(p):
            pltpu.make_async_copy(msg_bufs[p], acc.at[pl.ds(0, blk)],
                                  ssems[p]).wait()

        def compute(bi, p):
            rows_v = rows_bufs[p]
            msg_v = msg_bufs[p]
            woff = bi * blk
            for g in range(blk // 16):
                w0vec = w0_c[pl.ds(woff + g * 16, 16)]
                w1vec = w1_c[pl.ds(woff + g * 16, 16)]
                for j in range(16):
                    e = g * 16 + j
                    w0 = w0vec[j]
                    w1 = w1vec[j]
                    for f in range(n_chunks):
                        a = rows_v[e, pl.ds(16 * f, 16)]
                        bb = rows_v[e, pl.ds(f_half + 16 * f, 16)]
                        msg_v[e, pl.ds(16 * f, 16)] = w0 * a + w1 * bb

        def do_chunk(ci, _):
            base = pl.multiple_of(base0 + ci * chunk, 8)
            pltpu.sync_copy(ei_hbm.at[pl.ds(base, chunk)], src_c)
            pltpu.sync_copy(ei_hbm.at[pl.ds(E + base, chunk)], dst_c)
            pltpu.sync_copy(w_hbm.at[pl.ds(w_row * E + base, chunk)], w0_c)
            pltpu.sync_copy(w_hbm.at[pl.ds((w_row + 1) * E + base, chunk)],
                            w1_c)

            issue_gather(0, 0)
            issue_gather(1, 1)

            def body(b, _):
                for p in range(2):
                    @pl.when(b % 2 == p)
                    def _run():
                        wait_gather(p)

                        @pl.when(b >= 2)
                        def _ws():
                            wait_scatter(p)
                        compute(b, p)
                        issue_scatter(b, p)

                        @pl.when(b + 2 < nblk)
                        def _ig():
                            issue_gather(b + 2, p)
                return 0
            lax.fori_loop(0, nblk, body, 0)

            wait_scatter(0)
            wait_scatter(1)
            return 0
        lax.fori_loop(0, nchunks_e, do_chunk, 0)

        plsc.subcore_barrier()
        pltpu.sync_copy(acc.at[pl.ds(row0, stripe)],
                        out_hbm.at[c, pl.ds(row0, stripe)])

        @pl.when(s == 0)
        def _copy_rem():
            pltpu.sync_copy(acc.at[pl.ds(rem_base, rem)],
                            out_hbm.at[c, pl.ds(rem_base, rem)])

    return sc_layer


def kernel(x, edge_index, edge_attr, Wp0, bp0, g0, mu0, sigma0, root0, bias0,
           Wp1, bp1, g1, mu1, sigma1, root1, bias1):
    ea_t = edge_attr.T

    z0, r0 = _tc_prep(x, g0, root0)
    wts = _tc_weights(ea_t, Wp0, bp0[:, None], mu0, sigma0,
                      Wp1, bp1[:, None], mu1, sigma1)

    ei_flat = edge_index.reshape(2 * E)
    w_flat = wts.reshape(4 * E)

    sc0 = _make_sc_layer(f_half=64, n_chunks=4, ones_col=4, w_row=0)
    part0 = sc0(z0, ei_flat, w_flat)

    g1e = jnp.concatenate(
        [g1[:, :40], jnp.zeros((64, 24), jnp.float32),
         g1[:, 40:], jnp.zeros((64, 24), jnp.float32)], axis=1)
    root1p = jnp.pad(root1, ((0, 0), (0, 8)))
    z1, r1e = _tc_combine(part0[0], part0[1], r0, bias0[None, :], g1e, root1p)

    sc1 = _make_sc_layer(f_half=64, n_chunks=3, ones_col=-1, w_row=2)
    part1 = sc1(z1, ei_flat, w_flat)

    bias1p = jnp.pad(bias1, (0, 8))[None, :]
    return _tc_final(part1[0], part1[1], r1e, bias1p)
